# Initial kernel scaffold; baseline (speedup 1.0000x reference)
import jax
import jax.numpy as jnp
from jax.experimental import pallas as pl


def kernel(feat, segment_ids):
    raise NotImplementedError("write your pallas kernel here")


# speedup vs baseline: 2.5984x; 2.5984x over previous
import functools

import jax
import jax.numpy as jnp
from jax import lax
from jax.experimental import pallas as pl
from jax.experimental.pallas import tpu as pltpu
from jax.experimental.pallas import tpu_sc as plsc

_N_ROWS = 100000
_D = 256
_N_SEG = 512
_NC = 2
_NS = 16
_NW = _NC * _NS
_ROWS_PER_W = _N_ROWS // _NW
_CHUNK = 125
_NCHUNK = _ROWS_PER_W // _CHUNK
_SEG_PER_TILE = _N_SEG // _NS


def _sc_partial_body(feat_hbm, ids_hbm, out_hbm, ids_v, buf_v, zrow_v, acc_sh):
    c = lax.axis_index("c")
    s = lax.axis_index("s")
    wid = c * _NS + s

    zero16 = jnp.zeros((16,), jnp.float32)

    def _zero(i, carry):
        r = i // (_D // 16)
        k = i % (_D // 16)
        zrow_v[r, pl.ds(k * 16, 16)] = zero16
        return carry

    lax.fori_loop(0, _SEG_PER_TILE * (_D // 16), _zero, 0)
    pltpu.sync_copy(zrow_v, acc_sh.at[pl.ds(s * _SEG_PER_TILE, _SEG_PER_TILE)])

    pltpu.sync_copy(ids_hbm.at[wid], ids_v)

    plsc.subcore_barrier()

    def _chunk(j, carry):
        row0 = wid * _ROWS_PER_W + j * _CHUNK
        pltpu.sync_copy(feat_hbm.at[pl.ds(row0, _CHUNK), :], buf_v)
        pltpu.sync_copy(buf_v, acc_sh.at[ids_v.at[j]], add=True)
        return carry

    lax.fori_loop(0, _NCHUNK, _chunk, 0)

    plsc.subcore_barrier()
    pltpu.sync_copy(
        acc_sh.at[pl.ds(s * _SEG_PER_TILE, _SEG_PER_TILE)],
        out_hbm.at[c, pl.ds(s * _SEG_PER_TILE, _SEG_PER_TILE)],
    )


_sc_partial = pl.kernel(
    _sc_partial_body,
    out_type=jax.ShapeDtypeStruct((_NC, _N_SEG, _D), jnp.float32),
    mesh=plsc.VectorSubcoreMesh(core_axis_name="c", subcore_axis_name="s"),
    scratch_types=[
        pltpu.VMEM((_NCHUNK, _CHUNK), jnp.int32),
        pltpu.VMEM((_CHUNK, _D), jnp.float32),
        pltpu.VMEM((_SEG_PER_TILE, _D), jnp.float32),
        pltpu.VMEM_SHARED((_N_SEG, _D), jnp.float32),
    ],
    compiler_params=pltpu.CompilerParams(use_tc_tiling_on_sc=False),
)


def _combine_body(p_ref, o_ref):
    o_ref[...] = p_ref[0] + p_ref[1]


def _combine(partial):
    return pl.pallas_call(
        _combine_body,
        out_shape=jax.ShapeDtypeStruct((_N_SEG, _D), jnp.float32),
    )(partial)


@jax.jit
def kernel(feat, segment_ids):
    ids = segment_ids.astype(jnp.int32).reshape(_NW, _NCHUNK, _CHUNK)
    partial = _sc_partial(feat, ids)
    return _combine(partial)


# speedup vs baseline: 2.9793x; 1.1466x over previous
import functools

import jax
import jax.numpy as jnp
from jax import lax
from jax.experimental import pallas as pl
from jax.experimental.pallas import tpu as pltpu
from jax.experimental.pallas import tpu_sc as plsc

_N_ROWS = 100000
_D = 256
_N_SEG = 512
_NC = 2
_NS = 16
_NW = _NC * _NS
_ROWS_PER_W = _N_ROWS // _NW
_CHUNK = 125
_NCHUNK = _ROWS_PER_W // _CHUNK
_SEG_PER_TILE = _N_SEG // _NS


def _sc_partial_body(
    feat_hbm, ids_hbm, out_hbm, ids_v, buf_v, zrow_v, acc_sh, sem0, sem1
):
    c = lax.axis_index("c")
    s = lax.axis_index("s")
    wid = c * _NS + s

    zero16 = jnp.zeros((16,), jnp.float32)

    def _zero(i, carry):
        r = i // (_D // 16)
        k = i % (_D // 16)
        zrow_v[r, pl.ds(k * 16, 16)] = zero16
        return carry

    lax.fori_loop(0, _SEG_PER_TILE * (_D // 16), _zero, 0)
    pltpu.sync_copy(zrow_v, acc_sh.at[pl.ds(s * _SEG_PER_TILE, _SEG_PER_TILE)])

    pltpu.sync_copy(ids_hbm.at[wid], ids_v)

    plsc.subcore_barrier()

    def _src(j):
        row0 = wid * _ROWS_PER_W + j * _CHUNK
        return feat_hbm.at[pl.ds(row0, _CHUNK), :]

    pltpu.async_copy(_src(0), buf_v.at[0], sem0)

    def _pair(t, carry):
        j0 = 2 * t
        pltpu.async_copy(_src(j0 + 1), buf_v.at[1], sem1)
        pltpu.make_async_copy(_src(j0), buf_v.at[0], sem0).wait()
        pltpu.sync_copy(buf_v.at[0], acc_sh.at[ids_v.at[j0]], add=True)
        pltpu.async_copy(_src(j0 + 2), buf_v.at[0], sem0)
        pltpu.make_async_copy(_src(j0 + 1), buf_v.at[1], sem1).wait()
        pltpu.sync_copy(buf_v.at[1], acc_sh.at[ids_v.at[j0 + 1]], add=True)
        return carry

    lax.fori_loop(0, (_NCHUNK - 1) // 2, _pair, 0)
    pltpu.make_async_copy(_src(_NCHUNK - 1), buf_v.at[0], sem0).wait()
    pltpu.sync_copy(buf_v.at[0], acc_sh.at[ids_v.at[_NCHUNK - 1]], add=True)

    plsc.subcore_barrier()
    pltpu.sync_copy(
        acc_sh.at[pl.ds(s * _SEG_PER_TILE, _SEG_PER_TILE)],
        out_hbm.at[c, pl.ds(s * _SEG_PER_TILE, _SEG_PER_TILE)],
    )


_sc_partial = pl.kernel(
    _sc_partial_body,
    out_type=jax.ShapeDtypeStruct((_NC, _N_SEG, _D), jnp.float32),
    mesh=plsc.VectorSubcoreMesh(core_axis_name="c", subcore_axis_name="s"),
    scratch_types=[
        pltpu.VMEM((_NCHUNK, _CHUNK), jnp.int32),
        pltpu.VMEM((2, _CHUNK, _D), jnp.float32),
        pltpu.VMEM((_SEG_PER_TILE, _D), jnp.float32),
        pltpu.VMEM_SHARED((_N_SEG, _D), jnp.float32),
        pltpu.SemaphoreType.DMA,
        pltpu.SemaphoreType.DMA,
    ],
    compiler_params=pltpu.CompilerParams(use_tc_tiling_on_sc=False),
)


def _combine_body(p_ref, o_ref):
    o_ref[...] = p_ref[0] + p_ref[1]


def _combine(partial):
    return pl.pallas_call(
        _combine_body,
        out_shape=jax.ShapeDtypeStruct((_N_SEG, _D), jnp.float32),
    )(partial)


@jax.jit
def kernel(feat, segment_ids):
    ids = segment_ids.astype(jnp.int32).reshape(_NW, _NCHUNK, _CHUNK)
    partial = _sc_partial(feat, ids)
    return _combine(partial)
